# Initial kernel scaffold; baseline (speedup 1.0000x reference)
import jax
import jax.numpy as jnp
from jax.experimental import pallas as pl


def kernel(prot_x, prot_edge_index, prot_dist, prot_batch, W1, b1, W_out, b_out):
    raise NotImplementedError("write your pallas kernel here")


# speedup vs baseline: 46.1814x; 46.1814x over previous
import jax
import jax.numpy as jnp
from jax import lax
from jax.experimental import pallas as pl
from jax.experimental.pallas import tpu as pltpu
from jax.experimental.pallas import tpu_sc as plsc

N = 100000
E = 3200000
FEAT = 20
HALF = FEAT // 2
HALFW = 16
G = 64
OUT = 5

BLK = 1024
NB = 98
N_PAD = NB * BLK
CHUNK = 128
K_SUB = 8
NCH = E // CHUNK
NSUP = NCH // K_SUB
ROWS_PER_TILE = N_PAD // 16

_SC_PARAMS = pltpu.CompilerParams(use_tc_tiling_on_sc=False)


def _mesh():
    return plsc.VectorSubcoreMesh(core_axis_name="c", subcore_axis_name="s")


def _half_range(t):
    q, r = NSUP // 16, NSUP % 16
    base = t * q + jnp.minimum(t, r)
    n = q + jnp.where(t < r, 1, 0)
    return base, n


def _split_range(c, t):
    q, r = NSUP // 32, NSUP % 32
    w = c * 16 + t
    base = w * q + jnp.minimum(w, r)
    n = q + jnp.where(w < r, 1, 0)
    return base, n


def _deg_body(dst2_ref, zeros1_ref, out_ref, deg_sp, idx_v, ones_v):
    c = lax.axis_index("c")
    t = lax.axis_index("s")
    r0 = t * ROWS_PER_TILE
    for k in range(CHUNK // 16):
        ones_v[pl.ds(k * 16, 16)] = jnp.ones((16,), jnp.float32)
    pltpu.sync_copy(zeros1_ref.at[pl.ds(r0, ROWS_PER_TILE)],
                    deg_sp.at[pl.ds(r0, ROWS_PER_TILE)])
    plsc.subcore_barrier()
    base, n = _split_range(c, t)

    def sbody(s, carry):
        b8 = (base + s) * K_SUB
        pltpu.sync_copy(dst2_ref.at[pl.ds(b8, K_SUB)], idx_v)
        for j in range(K_SUB):
            pltpu.sync_copy(ones_v, deg_sp.at[idx_v.at[j]], add=True)
        return carry

    lax.fori_loop(0, n, sbody, 0)
    plsc.subcore_barrier()
    pltpu.sync_copy(deg_sp.at[pl.ds(r0, ROWS_PER_TILE)],
                    out_ref.at[c, pl.ds(r0, ROWS_PER_TILE)])


def _deg_call(dst2, zeros1):
    return pl.kernel(
        _deg_body,
        out_type=jax.ShapeDtypeStruct((2, N_PAD), jnp.float32),
        mesh=_mesh(),
        compiler_params=_SC_PARAMS,
        scratch_types=[
            pltpu.VMEM_SHARED((N_PAD,), jnp.float32),
            pltpu.VMEM((K_SUB, CHUNK), jnp.int32),
            pltpu.VMEM((CHUNK,), jnp.float32),
        ],
    )(dst2, zeros1)


def _scat_body(src2_ref, dst2_ref, y2_ref, zeros16_ref, part_ref,
               acc_sp, sidx_v, didx_v, rows_v, sem):
    c = lax.axis_index("c")
    t = lax.axis_index("s")
    r0 = t * ROWS_PER_TILE
    pltpu.sync_copy(zeros16_ref.at[pl.ds(r0, ROWS_PER_TILE)],
                    acc_sp.at[pl.ds(r0, ROWS_PER_TILE)])
    plsc.subcore_barrier()
    base, n = _half_range(t)

    def sbody(s, carry):
        b8 = (base + s) * K_SUB
        pltpu.sync_copy(src2_ref.at[pl.ds(b8, K_SUB)], sidx_v)
        pltpu.sync_copy(dst2_ref.at[pl.ds(b8, K_SUB)], didx_v)
        cps = [pltpu.async_copy(y2_ref.at[c].at[sidx_v.at[j]], rows_v.at[j],
                                sem)
               for j in range(K_SUB)]
        for cp in cps:
            cp.wait()
        for j in range(K_SUB):
            pltpu.sync_copy(rows_v.at[j], acc_sp.at[didx_v.at[j]], add=True)
        return carry

    lax.fori_loop(0, n, sbody, 0)
    plsc.subcore_barrier()
    pltpu.sync_copy(acc_sp.at[pl.ds(r0, ROWS_PER_TILE)],
                    part_ref.at[c, pl.ds(r0, ROWS_PER_TILE)])


def _scat_call(src2, dst2, y2, zeros16):
    return pl.kernel(
        _scat_body,
        out_type=jax.ShapeDtypeStruct((2, N_PAD, HALFW), jnp.float32),
        mesh=_mesh(),
        compiler_params=_SC_PARAMS,
        scratch_types=[
            pltpu.VMEM_SHARED((N_PAD, HALFW), jnp.float32),
            pltpu.VMEM((K_SUB, CHUNK), jnp.int32),
            pltpu.VMEM((K_SUB, CHUNK), jnp.int32),
            pltpu.VMEM((K_SUB, CHUNK, HALFW), jnp.float32),
            pltpu.SemaphoreType.DMA,
        ],
    )(src2, dst2, y2, zeros16)


def _y_body(x_ref, w1_ref, deg_ref, y_ref):
    xw = jnp.dot(x_ref[...], w1_ref[...], preferred_element_type=jnp.float32,
                 precision=lax.Precision.HIGHEST)
    deg = deg_ref[...]
    degsum = deg[0, 0] + deg[1, 0]
    dinv = lax.rsqrt(degsum + 1.0)
    y = xw * dinv
    zpad = jnp.zeros((BLK, HALFW - HALF), jnp.float32)
    y_ref[0] = jnp.concatenate([y[:, :HALF], zpad], axis=1)
    y_ref[1] = jnp.concatenate([y[:, HALF:], zpad], axis=1)


def _y_call(x, w1, deg4):
    return pl.pallas_call(
        _y_body,
        grid=(NB,),
        in_specs=[
            pl.BlockSpec((BLK, FEAT), lambda i: (i, 0)),
            pl.BlockSpec((FEAT, FEAT), lambda i: (0, 0)),
            pl.BlockSpec((2, 1, BLK, 1), lambda i: (0, i, 0, 0)),
        ],
        out_specs=pl.BlockSpec((2, BLK, HALFW), lambda i: (0, i, 0)),
        out_shape=jax.ShapeDtypeStruct((2, N_PAD, HALFW), jnp.float32),
    )(x, w1, deg4)


def _final_body(part_ref, y_ref, deg_ref, batch_ref, b1_ref, wout_ref,
                bout_ref, out_ref, s_acc):
    i = pl.program_id(0)
    part = part_ref[...]
    acc = jnp.concatenate([part[0, :, :HALF], part[1, :, :HALF]], axis=1)
    yb = y_ref[...]
    y = jnp.concatenate([yb[0, :, :HALF], yb[1, :, :HALF]], axis=1)
    deg = deg_ref[...]
    degsum = deg[0, 0] + deg[1, 0]
    dinv = lax.rsqrt(degsum + 1.0)
    h = (acc + y) * dinv + b1_ref[...]
    h = jnp.maximum(h, 0.0)
    rows = lax.broadcasted_iota(jnp.int32, (BLK, 1), 0) + i * BLK
    mask = rows < N
    h = jnp.where(mask, h, 0.0)
    b = batch_ref[...][0]
    oh = jnp.where(b == lax.broadcasted_iota(jnp.int32, (BLK, G), 1),
                   1.0, 0.0)
    hext = jnp.concatenate([h, mask.astype(jnp.float32)], axis=1)
    partial = lax.dot_general(oh, hext, (((0,), (0,)), ((), ())),
                              preferred_element_type=jnp.float32,
                              precision=lax.Precision.HIGHEST)
    prev = jnp.where(i == 0, jnp.zeros_like(partial), s_acc[...])
    tot = prev + partial
    s_acc[...] = tot

    @pl.when(i == NB - 1)
    def _():
        cnt = tot[:, FEAT:FEAT + 1]
        pooled = tot[:, :FEAT] / jnp.maximum(cnt, 1.0)
        logits = jnp.dot(pooled, wout_ref[...],
                         preferred_element_type=jnp.float32,
                         precision=lax.Precision.HIGHEST) + bout_ref[...]
        m = jnp.max(logits, axis=1, keepdims=True)
        ex = jnp.exp(logits - m)
        out_ref[...] = ex / jnp.sum(ex, axis=1, keepdims=True)


def _final_call(part, y2, deg4, batch3, b1, wout, bout):
    return pl.pallas_call(
        _final_body,
        grid=(NB,),
        in_specs=[
            pl.BlockSpec((2, BLK, HALFW), lambda i: (0, i, 0)),
            pl.BlockSpec((2, BLK, HALFW), lambda i: (0, i, 0)),
            pl.BlockSpec((2, 1, BLK, 1), lambda i: (0, i, 0, 0)),
            pl.BlockSpec((1, BLK, 1), lambda i: (i, 0, 0)),
            pl.BlockSpec((1, FEAT), lambda i: (0, 0)),
            pl.BlockSpec((FEAT, OUT), lambda i: (0, 0)),
            pl.BlockSpec((1, OUT), lambda i: (0, 0)),
        ],
        out_specs=pl.BlockSpec((G, OUT), lambda i: (0, 0)),
        out_shape=jax.ShapeDtypeStruct((G, OUT), jnp.float32),
        scratch_shapes=[pltpu.VMEM((G, FEAT + 1), jnp.float32)],
    )(part, y2, deg4, batch3, b1, wout, bout)


def kernel(prot_x, prot_edge_index, prot_dist, prot_batch, W1, b1, W_out,
           b_out):
    del prot_dist
    src2 = prot_edge_index[0].reshape(NCH, CHUNK)
    dst2 = prot_edge_index[1].reshape(NCH, CHUNK)
    zeros1 = jnp.zeros((N_PAD,), jnp.float32)
    zeros16 = jnp.zeros((N_PAD, HALFW), jnp.float32)

    degp = _deg_call(dst2, zeros1)
    deg4 = degp.reshape(2, NB, BLK, 1)
    y2 = _y_call(prot_x, W1, deg4)
    part = _scat_call(src2, dst2, y2, zeros16)

    batch3 = jnp.concatenate(
        [prot_batch, jnp.full((N_PAD - N,), G, jnp.int32)]).reshape(NB, BLK, 1)
    return _final_call(part, y2, deg4, batch3, b1.reshape(1, FEAT), W_out,
                       b_out.reshape(1, OUT))


# speedup vs baseline: 53.7281x; 1.1634x over previous
import jax
import jax.numpy as jnp
from jax import lax
from jax.experimental import pallas as pl
from jax.experimental.pallas import tpu as pltpu
from jax.experimental.pallas import tpu_sc as plsc

N = 100000
E = 3200000
FEAT = 20
HALF = FEAT // 2
HALFW = 16
G = 64
OUT = 5

BLK = 1024
NB = 98
N_PAD = NB * BLK
CHUNK = 128
K_SUB = 8
NCH = E // CHUNK
NSUP = NCH // K_SUB
ROWS_PER_TILE = N_PAD // 16

_SC_PARAMS = pltpu.CompilerParams(use_tc_tiling_on_sc=False)


def _mesh():
    return plsc.VectorSubcoreMesh(core_axis_name="c", subcore_axis_name="s")


def _half_range(t):
    q, r = NSUP // 16, NSUP % 16
    base = t * q + jnp.minimum(t, r)
    n = q + jnp.where(t < r, 1, 0)
    return base, n


def _split_range(c, t):
    q, r = NSUP // 32, NSUP % 32
    w = c * 16 + t
    base = w * q + jnp.minimum(w, r)
    n = q + jnp.where(w < r, 1, 0)
    return base, n


def _deg_body(dst2_ref, zeros1_ref, out_ref, deg_sp, idx_v, ones_v):
    c = lax.axis_index("c")
    t = lax.axis_index("s")
    r0 = t * ROWS_PER_TILE
    for k in range(CHUNK // 16):
        ones_v[pl.ds(k * 16, 16)] = jnp.ones((16,), jnp.float32)
    pltpu.sync_copy(zeros1_ref.at[pl.ds(r0, ROWS_PER_TILE)],
                    deg_sp.at[pl.ds(r0, ROWS_PER_TILE)])
    plsc.subcore_barrier()
    base, n = _split_range(c, t)

    def sbody(s, carry):
        b8 = (base + s) * K_SUB
        pltpu.sync_copy(dst2_ref.at[pl.ds(b8, K_SUB)], idx_v)
        for j in range(K_SUB):
            pltpu.sync_copy(ones_v, deg_sp.at[idx_v.at[j]], add=True)
        return carry

    lax.fori_loop(0, n, sbody, 0)
    plsc.subcore_barrier()
    pltpu.sync_copy(deg_sp.at[pl.ds(r0, ROWS_PER_TILE)],
                    out_ref.at[c, pl.ds(r0, ROWS_PER_TILE)])


def _deg_call(dst2, zeros1):
    return pl.kernel(
        _deg_body,
        out_type=jax.ShapeDtypeStruct((2, N_PAD), jnp.float32),
        mesh=_mesh(),
        compiler_params=_SC_PARAMS,
        scratch_types=[
            pltpu.VMEM_SHARED((N_PAD,), jnp.float32),
            pltpu.VMEM((K_SUB, CHUNK), jnp.int32),
            pltpu.VMEM((CHUNK,), jnp.float32),
        ],
    )(dst2, zeros1)


K_E = 4
NSUP_E = NCH // K_E
NBUF = 3


def _scat_body(src2_ref, dst2_ref, y2_ref, zeros16_ref, part_ref,
               acc_sp, sidx_v, didx_v, rows_v, sg0, sg1, sg2, ss0, ss1, ss2):
    c = lax.axis_index("c")
    t = lax.axis_index("s")
    r0 = t * ROWS_PER_TILE
    pltpu.sync_copy(zeros16_ref.at[pl.ds(r0, ROWS_PER_TILE)],
                    acc_sp.at[pl.ds(r0, ROWS_PER_TILE)])
    plsc.subcore_barrier()
    q, r = NSUP_E // 16, NSUP_E % 16
    base = t * q + jnp.minimum(t, r)
    n = q + jnp.where(t < r, 1, 0)
    sg = (sg0, sg1, sg2)
    ss = (ss0, ss1, ss2)

    def fire(su, b):
        bk = su * K_E
        pltpu.sync_copy(src2_ref.at[pl.ds(bk, K_E)], sidx_v.at[b])
        pltpu.sync_copy(dst2_ref.at[pl.ds(bk, K_E)], didx_v.at[b])
        for j in range(K_E):
            pltpu.async_copy(y2_ref.at[c].at[sidx_v.at[b].at[j]],
                             rows_v.at[b, j], sg[b])

    def wait_gathers(b):
        for j in range(K_E):
            pltpu.make_async_copy(zeros16_ref.at[pl.ds(0, CHUNK)],
                                  rows_v.at[b, j], sg[b]).wait()

    def fire_scatter(b):
        for j in range(K_E):
            pltpu.async_copy(rows_v.at[b, j], acc_sp.at[didx_v.at[b].at[j]],
                             ss[b], add=True)

    def drain_scatter(b):
        for j in range(K_E):
            pltpu.make_async_copy(zeros16_ref.at[pl.ds(0, CHUNK)],
                                  rows_v.at[b, j], ss[b]).wait()

    fire(base, 0)
    fire(base + 1, 1)

    def sbody(s, carry):
        for P in range(NBUF):
            @pl.when(lax.rem(s, NBUF) == P)
            def _(P=P):
                @pl.when(s >= 1)
                def _():
                    drain_scatter((P + NBUF - 1) % NBUF)

                @pl.when(s + 2 <= n - 1)
                def _():
                    fire(base + s + 2, (P + 2) % NBUF)

                wait_gathers(P)
                fire_scatter(P)
        return carry

    lax.fori_loop(0, n, sbody, 0)
    for P in range(NBUF):
        @pl.when(lax.rem(n - 1, NBUF) == P)
        def _(P=P):
            drain_scatter(P)
    plsc.subcore_barrier()
    pltpu.sync_copy(acc_sp.at[pl.ds(r0, ROWS_PER_TILE)],
                    part_ref.at[c, pl.ds(r0, ROWS_PER_TILE)])


def _scat_call(src2, dst2, y2, zeros16):
    return pl.kernel(
        _scat_body,
        out_type=jax.ShapeDtypeStruct((2, N_PAD, HALFW), jnp.float32),
        mesh=_mesh(),
        compiler_params=_SC_PARAMS,
        scratch_types=[
            pltpu.VMEM_SHARED((N_PAD, HALFW), jnp.float32),
            pltpu.VMEM((NBUF, K_E, CHUNK), jnp.int32),
            pltpu.VMEM((NBUF, K_E, CHUNK), jnp.int32),
            pltpu.VMEM((NBUF, K_E, CHUNK, HALFW), jnp.float32),
            pltpu.SemaphoreType.DMA,
            pltpu.SemaphoreType.DMA,
            pltpu.SemaphoreType.DMA,
            pltpu.SemaphoreType.DMA,
            pltpu.SemaphoreType.DMA,
            pltpu.SemaphoreType.DMA,
        ],
    )(src2, dst2, y2, zeros16)


def _y_body(x_ref, w1_ref, deg_ref, y_ref):
    xw = jnp.dot(x_ref[...], w1_ref[...], preferred_element_type=jnp.float32,
                 precision=lax.Precision.HIGHEST)
    deg = deg_ref[...]
    degsum = deg[0, 0] + deg[1, 0]
    dinv = lax.rsqrt(degsum + 1.0)
    y = xw * dinv
    zpad = jnp.zeros((BLK, HALFW - HALF), jnp.float32)
    y_ref[0] = jnp.concatenate([y[:, :HALF], zpad], axis=1)
    y_ref[1] = jnp.concatenate([y[:, HALF:], zpad], axis=1)


def _y_call(x, w1, deg4):
    return pl.pallas_call(
        _y_body,
        grid=(NB,),
        in_specs=[
            pl.BlockSpec((BLK, FEAT), lambda i: (i, 0)),
            pl.BlockSpec((FEAT, FEAT), lambda i: (0, 0)),
            pl.BlockSpec((2, 1, BLK, 1), lambda i: (0, i, 0, 0)),
        ],
        out_specs=pl.BlockSpec((2, BLK, HALFW), lambda i: (0, i, 0)),
        out_shape=jax.ShapeDtypeStruct((2, N_PAD, HALFW), jnp.float32),
    )(x, w1, deg4)


def _final_body(part_ref, y_ref, deg_ref, batch_ref, b1_ref, wout_ref,
                bout_ref, out_ref, s_acc):
    i = pl.program_id(0)
    part = part_ref[...]
    acc = jnp.concatenate([part[0, :, :HALF], part[1, :, :HALF]], axis=1)
    yb = y_ref[...]
    y = jnp.concatenate([yb[0, :, :HALF], yb[1, :, :HALF]], axis=1)
    deg = deg_ref[...]
    degsum = deg[0, 0] + deg[1, 0]
    dinv = lax.rsqrt(degsum + 1.0)
    h = (acc + y) * dinv + b1_ref[...]
    h = jnp.maximum(h, 0.0)
    rows = lax.broadcasted_iota(jnp.int32, (BLK, 1), 0) + i * BLK
    mask = rows < N
    h = jnp.where(mask, h, 0.0)
    b = batch_ref[...][0]
    oh = jnp.where(b == lax.broadcasted_iota(jnp.int32, (BLK, G), 1),
                   1.0, 0.0)
    hext = jnp.concatenate([h, mask.astype(jnp.float32)], axis=1)
    partial = lax.dot_general(oh, hext, (((0,), (0,)), ((), ())),
                              preferred_element_type=jnp.float32,
                              precision=lax.Precision.HIGHEST)
    prev = jnp.where(i == 0, jnp.zeros_like(partial), s_acc[...])
    tot = prev + partial
    s_acc[...] = tot

    @pl.when(i == NB - 1)
    def _():
        cnt = tot[:, FEAT:FEAT + 1]
        pooled = tot[:, :FEAT] / jnp.maximum(cnt, 1.0)
        logits = jnp.dot(pooled, wout_ref[...],
                         preferred_element_type=jnp.float32,
                         precision=lax.Precision.HIGHEST) + bout_ref[...]
        m = jnp.max(logits, axis=1, keepdims=True)
        ex = jnp.exp(logits - m)
        out_ref[...] = ex / jnp.sum(ex, axis=1, keepdims=True)


def _final_call(part, y2, deg4, batch3, b1, wout, bout):
    return pl.pallas_call(
        _final_body,
        grid=(NB,),
        in_specs=[
            pl.BlockSpec((2, BLK, HALFW), lambda i: (0, i, 0)),
            pl.BlockSpec((2, BLK, HALFW), lambda i: (0, i, 0)),
            pl.BlockSpec((2, 1, BLK, 1), lambda i: (0, i, 0, 0)),
            pl.BlockSpec((1, BLK, 1), lambda i: (i, 0, 0)),
            pl.BlockSpec((1, FEAT), lambda i: (0, 0)),
            pl.BlockSpec((FEAT, OUT), lambda i: (0, 0)),
            pl.BlockSpec((1, OUT), lambda i: (0, 0)),
        ],
        out_specs=pl.BlockSpec((G, OUT), lambda i: (0, 0)),
        out_shape=jax.ShapeDtypeStruct((G, OUT), jnp.float32),
        scratch_shapes=[pltpu.VMEM((G, FEAT + 1), jnp.float32)],
    )(part, y2, deg4, batch3, b1, wout, bout)


def kernel(prot_x, prot_edge_index, prot_dist, prot_batch, W1, b1, W_out,
           b_out):
    del prot_dist
    src2 = prot_edge_index[0].reshape(NCH, CHUNK)
    dst2 = prot_edge_index[1].reshape(NCH, CHUNK)
    zeros1 = jnp.zeros((N_PAD,), jnp.float32)
    zeros16 = jnp.zeros((N_PAD, HALFW), jnp.float32)

    degp = _deg_call(dst2, zeros1)
    deg4 = degp.reshape(2, NB, BLK, 1)
    y2 = _y_call(prot_x, W1, deg4)
    part = _scat_call(src2, dst2, y2, zeros16)

    batch3 = jnp.concatenate(
        [prot_batch, jnp.full((N_PAD - N,), G, jnp.int32)]).reshape(NB, BLK, 1)
    return _final_call(part, y2, deg4, batch3, b1.reshape(1, FEAT), W_out,
                       b_out.reshape(1, OUT))


# speedup vs baseline: 58.0032x; 1.0796x over previous
import jax
import jax.numpy as jnp
from jax import lax
from jax.experimental import pallas as pl
from jax.experimental.pallas import tpu as pltpu
from jax.experimental.pallas import tpu_sc as plsc

N = 100000
E = 3200000
FEAT = 20
HALF = FEAT // 2
HALFW = 16
G = 64
OUT = 5

BLK = 6272
NB = 16
N_PAD = NB * BLK
CHUNK = 128
K_SUB = 8
NCH = E // CHUNK
NSUP = NCH // K_SUB
ROWS_PER_TILE = N_PAD // 16

_SC_PARAMS = pltpu.CompilerParams(use_tc_tiling_on_sc=False)


def _mesh():
    return plsc.VectorSubcoreMesh(core_axis_name="c", subcore_axis_name="s")


def _half_range(t):
    q, r = NSUP // 16, NSUP % 16
    base = t * q + jnp.minimum(t, r)
    n = q + jnp.where(t < r, 1, 0)
    return base, n


def _split_range(c, t):
    q, r = NSUP // 32, NSUP % 32
    w = c * 16 + t
    base = w * q + jnp.minimum(w, r)
    n = q + jnp.where(w < r, 1, 0)
    return base, n


def _deg_body(dst2_ref, zeros1_ref, out_ref, deg_sp, idx_v, ones_v):
    c = lax.axis_index("c")
    t = lax.axis_index("s")
    r0 = t * ROWS_PER_TILE
    for k in range(CHUNK // 16):
        ones_v[pl.ds(k * 16, 16)] = jnp.ones((16,), jnp.float32)
    pltpu.sync_copy(zeros1_ref.at[pl.ds(r0, ROWS_PER_TILE)],
                    deg_sp.at[pl.ds(r0, ROWS_PER_TILE)])
    plsc.subcore_barrier()
    base, n = _split_range(c, t)

    def sbody(s, carry):
        b8 = (base + s) * K_SUB
        pltpu.sync_copy(dst2_ref.at[pl.ds(b8, K_SUB)], idx_v)
        for j in range(K_SUB):
            pltpu.sync_copy(ones_v, deg_sp.at[idx_v.at[j]], add=True)
        return carry

    lax.fori_loop(0, n, sbody, 0)
    plsc.subcore_barrier()
    pltpu.sync_copy(deg_sp.at[pl.ds(r0, ROWS_PER_TILE)],
                    out_ref.at[c, pl.ds(r0, ROWS_PER_TILE)])


def _deg_call(dst2, zeros1):
    return pl.kernel(
        _deg_body,
        out_type=jax.ShapeDtypeStruct((2, N_PAD), jnp.float32),
        mesh=_mesh(),
        compiler_params=_SC_PARAMS,
        scratch_types=[
            pltpu.VMEM_SHARED((N_PAD,), jnp.float32),
            pltpu.VMEM((K_SUB, CHUNK), jnp.int32),
            pltpu.VMEM((CHUNK,), jnp.float32),
        ],
    )(dst2, zeros1)


K_E = 4
NSUP_E = NCH // K_E
NBUF = 3


def _scat_body(src2_ref, dst2_ref, y2_ref, zeros16_ref, part_ref,
               acc_sp, sidx_v, didx_v, rows_v, sg0, sg1, sg2, ss0, ss1, ss2):
    c = lax.axis_index("c")
    t = lax.axis_index("s")
    r0 = t * ROWS_PER_TILE
    pltpu.sync_copy(zeros16_ref.at[pl.ds(r0, ROWS_PER_TILE)],
                    acc_sp.at[pl.ds(r0, ROWS_PER_TILE)])
    plsc.subcore_barrier()
    q, r = NSUP_E // 16, NSUP_E % 16
    base = t * q + jnp.minimum(t, r)
    n = q + jnp.where(t < r, 1, 0)
    sg = (sg0, sg1, sg2)
    ss = (ss0, ss1, ss2)

    def fire(su, b):
        bk = su * K_E
        pltpu.sync_copy(src2_ref.at[pl.ds(bk, K_E)], sidx_v.at[b])
        pltpu.sync_copy(dst2_ref.at[pl.ds(bk, K_E)], didx_v.at[b])
        for j in range(K_E):
            pltpu.async_copy(y2_ref.at[c].at[sidx_v.at[b].at[j]],
                             rows_v.at[b, j], sg[b])

    def wait_gathers(b):
        for j in range(K_E):
            pltpu.make_async_copy(zeros16_ref.at[pl.ds(0, CHUNK)],
                                  rows_v.at[b, j], sg[b]).wait()

    def fire_scatter(b):
        for j in range(K_E):
            pltpu.async_copy(rows_v.at[b, j], acc_sp.at[didx_v.at[b].at[j]],
                             ss[b], add=True)

    def drain_scatter(b):
        for j in range(K_E):
            pltpu.make_async_copy(zeros16_ref.at[pl.ds(0, CHUNK)],
                                  rows_v.at[b, j], ss[b]).wait()

    fire(base, 0)
    fire(base + 1, 1)

    def sbody(s, carry):
        for P in range(NBUF):
            @pl.when(lax.rem(s, NBUF) == P)
            def _(P=P):
                @pl.when(s >= 1)
                def _():
                    drain_scatter((P + NBUF - 1) % NBUF)

                @pl.when(s + 2 <= n - 1)
                def _():
                    fire(base + s + 2, (P + 2) % NBUF)

                wait_gathers(P)
                fire_scatter(P)
        return carry

    lax.fori_loop(0, n, sbody, 0)
    for P in range(NBUF):
        @pl.when(lax.rem(n - 1, NBUF) == P)
        def _(P=P):
            drain_scatter(P)
    plsc.subcore_barrier()
    pltpu.sync_copy(acc_sp.at[pl.ds(r0, ROWS_PER_TILE)],
                    part_ref.at[c, pl.ds(r0, ROWS_PER_TILE)])


def _scat_call(src2, dst2, y2, zeros16):
    return pl.kernel(
        _scat_body,
        out_type=jax.ShapeDtypeStruct((2, N_PAD, HALFW), jnp.float32),
        mesh=_mesh(),
        compiler_params=_SC_PARAMS,
        scratch_types=[
            pltpu.VMEM_SHARED((N_PAD, HALFW), jnp.float32),
            pltpu.VMEM((NBUF, K_E, CHUNK), jnp.int32),
            pltpu.VMEM((NBUF, K_E, CHUNK), jnp.int32),
            pltpu.VMEM((NBUF, K_E, CHUNK, HALFW), jnp.float32),
            pltpu.SemaphoreType.DMA,
            pltpu.SemaphoreType.DMA,
            pltpu.SemaphoreType.DMA,
            pltpu.SemaphoreType.DMA,
            pltpu.SemaphoreType.DMA,
            pltpu.SemaphoreType.DMA,
        ],
    )(src2, dst2, y2, zeros16)


def _y_body(x_ref, w1_ref, deg_ref, y_ref):
    xw = jnp.dot(x_ref[...], w1_ref[...], preferred_element_type=jnp.float32,
                 precision=lax.Precision.HIGHEST)
    deg = deg_ref[...]
    degsum = deg[0, 0] + deg[1, 0]
    dinv = lax.rsqrt(degsum + 1.0)
    y = xw * dinv
    zpad = jnp.zeros((BLK, HALFW - HALF), jnp.float32)
    y_ref[0] = jnp.concatenate([y[:, :HALF], zpad], axis=1)
    y_ref[1] = jnp.concatenate([y[:, HALF:], zpad], axis=1)


def _y_call(x, w1, deg4):
    return pl.pallas_call(
        _y_body,
        grid=(NB,),
        in_specs=[
            pl.BlockSpec((BLK, FEAT), lambda i: (i, 0)),
            pl.BlockSpec((FEAT, FEAT), lambda i: (0, 0)),
            pl.BlockSpec((2, 1, BLK, 1), lambda i: (0, i, 0, 0)),
        ],
        out_specs=pl.BlockSpec((2, BLK, HALFW), lambda i: (0, i, 0)),
        out_shape=jax.ShapeDtypeStruct((2, N_PAD, HALFW), jnp.float32),
    )(x, w1, deg4)


def _final_body(part_ref, y_ref, deg_ref, batch_ref, b1_ref, wout_ref,
                bout_ref, out_ref, s_acc):
    i = pl.program_id(0)
    part = part_ref[...]
    acc = jnp.concatenate([part[0, :, :HALF], part[1, :, :HALF]], axis=1)
    yb = y_ref[...]
    y = jnp.concatenate([yb[0, :, :HALF], yb[1, :, :HALF]], axis=1)
    deg = deg_ref[...]
    degsum = deg[0, 0] + deg[1, 0]
    dinv = lax.rsqrt(degsum + 1.0)
    h = (acc + y) * dinv + b1_ref[...]
    h = jnp.maximum(h, 0.0)
    rows = lax.broadcasted_iota(jnp.int32, (BLK, 1), 0) + i * BLK
    mask = rows < N
    h = jnp.where(mask, h, 0.0)
    b = batch_ref[...][0]
    oh = jnp.where(b == lax.broadcasted_iota(jnp.int32, (BLK, G), 1),
                   1.0, 0.0)
    hext = jnp.concatenate([h, mask.astype(jnp.float32)], axis=1)
    partial = lax.dot_general(oh, hext, (((0,), (0,)), ((), ())),
                              preferred_element_type=jnp.float32,
                              precision=lax.Precision.HIGHEST)
    prev = jnp.where(i == 0, jnp.zeros_like(partial), s_acc[...])
    tot = prev + partial
    s_acc[...] = tot

    @pl.when(i == NB - 1)
    def _():
        cnt = tot[:, FEAT:FEAT + 1]
        pooled = tot[:, :FEAT] / jnp.maximum(cnt, 1.0)
        logits = jnp.dot(pooled, wout_ref[...],
                         preferred_element_type=jnp.float32,
                         precision=lax.Precision.HIGHEST) + bout_ref[...]
        m = jnp.max(logits, axis=1, keepdims=True)
        ex = jnp.exp(logits - m)
        out_ref[...] = ex / jnp.sum(ex, axis=1, keepdims=True)


def _final_call(part, y2, deg4, batch3, b1, wout, bout):
    return pl.pallas_call(
        _final_body,
        grid=(NB,),
        in_specs=[
            pl.BlockSpec((2, BLK, HALFW), lambda i: (0, i, 0)),
            pl.BlockSpec((2, BLK, HALFW), lambda i: (0, i, 0)),
            pl.BlockSpec((2, 1, BLK, 1), lambda i: (0, i, 0, 0)),
            pl.BlockSpec((1, BLK, 1), lambda i: (i, 0, 0)),
            pl.BlockSpec((1, FEAT), lambda i: (0, 0)),
            pl.BlockSpec((FEAT, OUT), lambda i: (0, 0)),
            pl.BlockSpec((1, OUT), lambda i: (0, 0)),
        ],
        out_specs=pl.BlockSpec((G, OUT), lambda i: (0, 0)),
        out_shape=jax.ShapeDtypeStruct((G, OUT), jnp.float32),
        scratch_shapes=[pltpu.VMEM((G, FEAT + 1), jnp.float32)],
    )(part, y2, deg4, batch3, b1, wout, bout)


def kernel(prot_x, prot_edge_index, prot_dist, prot_batch, W1, b1, W_out,
           b_out):
    del prot_dist
    src2 = prot_edge_index[0].reshape(NCH, CHUNK)
    dst2 = prot_edge_index[1].reshape(NCH, CHUNK)
    zeros1 = jnp.zeros((N_PAD,), jnp.float32)
    zeros16 = jnp.zeros((N_PAD, HALFW), jnp.float32)

    degp = _deg_call(dst2, zeros1)
    deg4 = degp.reshape(2, NB, BLK, 1)
    y2 = _y_call(prot_x, W1, deg4)
    part = _scat_call(src2, dst2, y2, zeros16)

    batch3 = jnp.concatenate(
        [prot_batch, jnp.full((N_PAD - N,), G, jnp.int32)]).reshape(NB, BLK, 1)
    return _final_call(part, y2, deg4, batch3, b1.reshape(1, FEAT), W_out,
                       b_out.reshape(1, OUT))


# speedup vs baseline: 68.4923x; 1.1808x over previous
import jax
import jax.numpy as jnp
from jax import lax
from jax.experimental import pallas as pl
from jax.experimental.pallas import tpu as pltpu
from jax.experimental.pallas import tpu_sc as plsc

N = 100000
E = 3200000
FEAT = 20
HALF = FEAT // 2
HALFW = 16
G = 64
OUT = 5

BLK = 6272
NB = 16
N_PAD = NB * BLK
CHUNK = 128
K_SUB = 8
NCH = E // CHUNK
NSUP = NCH // K_SUB
ROWS_PER_TILE = N_PAD // 16

_SC_PARAMS = pltpu.CompilerParams(use_tc_tiling_on_sc=False)


def _mesh():
    return plsc.VectorSubcoreMesh(core_axis_name="c", subcore_axis_name="s")


def _half_range(t):
    q, r = NSUP // 16, NSUP % 16
    base = t * q + jnp.minimum(t, r)
    n = q + jnp.where(t < r, 1, 0)
    return base, n


def _split_range(c, t):
    q, r = NSUP // 32, NSUP % 32
    w = c * 16 + t
    base = w * q + jnp.minimum(w, r)
    n = q + jnp.where(w < r, 1, 0)
    return base, n


def _deg_body(dst2_ref, zeros1_ref, ones_ref, out_ref, deg_sp, idx_v, ones_v):
    c = lax.axis_index("c")
    t = lax.axis_index("s")
    r0 = t * ROWS_PER_TILE
    pltpu.sync_copy(ones_ref, ones_v)
    pltpu.sync_copy(zeros1_ref.at[pl.ds(r0, ROWS_PER_TILE)],
                    deg_sp.at[pl.ds(r0, ROWS_PER_TILE)])
    plsc.subcore_barrier()
    base, n = _split_range(c, t)

    def sbody(s, carry):
        b8 = (base + s) * K_SUB
        pltpu.sync_copy(dst2_ref.at[pl.ds(b8, K_SUB)], idx_v)
        for j in range(K_SUB):
            pltpu.sync_copy(ones_v, deg_sp.at[idx_v.at[j]], add=True)
        return carry

    lax.fori_loop(0, n, sbody, 0)
    plsc.subcore_barrier()
    pltpu.sync_copy(deg_sp.at[pl.ds(r0, ROWS_PER_TILE)], out_ref.at[c, t])


def _deg_call(dst2, zeros1, ones1):
    return pl.kernel(
        _deg_body,
        out_type=jax.ShapeDtypeStruct((2, NB, BLK, 1), jnp.float32),
        mesh=_mesh(),
        compiler_params=_SC_PARAMS,
        scratch_types=[
            pltpu.VMEM_SHARED((N_PAD, 1), jnp.float32),
            pltpu.VMEM((K_SUB, CHUNK), jnp.int32),
            pltpu.VMEM((CHUNK, 1), jnp.float32),
        ],
    )(dst2, zeros1, ones1)


K_E = 4
NSUP_E = NCH // K_E
NBUF = 3


def _scat_body(src2_ref, dst2_ref, y2_ref, zeros16_ref, part_ref,
               acc_sp, sidx_v, didx_v, rows_v,
               sg0, sg1, sg2, ss0, ss1, ss2, si0, si1, si2, sd):
    c = lax.axis_index("c")
    t = lax.axis_index("s")
    r0 = t * ROWS_PER_TILE
    pltpu.sync_copy(zeros16_ref.at[pl.ds(r0, ROWS_PER_TILE)],
                    acc_sp.at[pl.ds(r0, ROWS_PER_TILE)])
    plsc.subcore_barrier()
    q, r = NSUP_E // 16, NSUP_E % 16
    base = t * q + jnp.minimum(t, r)
    n = q + jnp.where(t < r, 1, 0)
    sg = (sg0, sg1, sg2)
    ss = (ss0, ss1, ss2)
    si = (si0, si1, si2)

    def fire_sidx(su, b):
        pltpu.async_copy(src2_ref.at[pl.ds(su * K_E, K_E)], sidx_v.at[b],
                         si[b])

    def fire_didx(su, b):
        pltpu.async_copy(dst2_ref.at[pl.ds(su * K_E, K_E)], didx_v.at[b], sd)

    def fire_gathers(su, b):
        pltpu.make_async_copy(src2_ref.at[pl.ds(0, K_E)], sidx_v.at[b],
                              si[b]).wait()
        for j in range(K_E):
            pltpu.async_copy(y2_ref.at[c].at[sidx_v.at[b].at[j]],
                             rows_v.at[b, j], sg[b])

    def wait_gathers(b):
        for j in range(K_E):
            pltpu.make_async_copy(zeros16_ref.at[pl.ds(0, CHUNK)],
                                  rows_v.at[b, j], sg[b]).wait()

    def wait_didx(b):
        pltpu.make_async_copy(src2_ref.at[pl.ds(0, K_E)], didx_v.at[b],
                              sd).wait()

    def fire_scatter(b):
        for j in range(K_E):
            pltpu.async_copy(rows_v.at[b, j], acc_sp.at[didx_v.at[b].at[j]],
                             ss[b], add=True)

    def drain_scatter(b):
        for j in range(K_E):
            pltpu.make_async_copy(zeros16_ref.at[pl.ds(0, CHUNK)],
                                  rows_v.at[b, j], ss[b]).wait()

    fire_sidx(base, 0)
    fire_sidx(base + 1, 1)
    fire_sidx(base + 2, 2)
    fire_gathers(base, 0)
    fire_gathers(base + 1, 1)

    def sbody(s, carry):
        for P in range(NBUF):
            @pl.when(lax.rem(s, NBUF) == P)
            def _(P=P):
                @pl.when(s >= 1)
                def _():
                    drain_scatter((P + 2) % NBUF)

                fire_didx(base + s, P)

                @pl.when(s + 2 <= n - 1)
                def _():
                    fire_gathers(base + s + 2, (P + 2) % NBUF)

                wait_gathers(P)
                wait_didx(P)
                fire_scatter(P)

                @pl.when(s + 3 <= n - 1)
                def _():
                    fire_sidx(base + s + 3, P)
        return carry

    lax.fori_loop(0, n, sbody, 0)
    for P in range(NBUF):
        @pl.when(lax.rem(n - 1, NBUF) == P)
        def _(P=P):
            drain_scatter(P)
    plsc.subcore_barrier()
    pltpu.sync_copy(acc_sp.at[pl.ds(r0, ROWS_PER_TILE)],
                    part_ref.at[c, pl.ds(r0, ROWS_PER_TILE)])


def _scat_call(src2, dst2, y2, zeros16):
    return pl.kernel(
        _scat_body,
        out_type=jax.ShapeDtypeStruct((2, N_PAD, HALFW), jnp.float32),
        mesh=_mesh(),
        compiler_params=_SC_PARAMS,
        scratch_types=[
            pltpu.VMEM_SHARED((N_PAD, HALFW), jnp.float32),
            pltpu.VMEM((NBUF, K_E, CHUNK), jnp.int32),
            pltpu.VMEM((NBUF, K_E, CHUNK), jnp.int32),
            pltpu.VMEM((NBUF, K_E, CHUNK, HALFW), jnp.float32),
        ] + [pltpu.SemaphoreType.DMA] * 10,
    )(src2, dst2, y2, zeros16)


def _y_body(x_ref, w1_ref, deg_ref, y_ref):
    xw = jnp.dot(x_ref[...], w1_ref[...], preferred_element_type=jnp.float32,
                 precision=lax.Precision.HIGHEST)
    deg = deg_ref[...]
    degsum = deg[0, 0] + deg[1, 0]
    dinv = lax.rsqrt(degsum + 1.0)
    y = xw * dinv
    zpad = jnp.zeros((BLK, HALFW - HALF), jnp.float32)
    y_ref[0] = jnp.concatenate([y[:, :HALF], zpad], axis=1)
    y_ref[1] = jnp.concatenate([y[:, HALF:], zpad], axis=1)


def _y_call(x, w1, deg4):
    return pl.pallas_call(
        _y_body,
        grid=(NB,),
        in_specs=[
            pl.BlockSpec((BLK, FEAT), lambda i: (i, 0)),
            pl.BlockSpec((FEAT, FEAT), lambda i: (0, 0)),
            pl.BlockSpec((2, 1, BLK, 1), lambda i: (0, i, 0, 0)),
        ],
        out_specs=pl.BlockSpec((2, BLK, HALFW), lambda i: (0, i, 0)),
        out_shape=jax.ShapeDtypeStruct((2, N_PAD, HALFW), jnp.float32),
    )(x, w1, deg4)


def _final_body(part_ref, y_ref, deg_ref, batch_ref, b1_ref, wout_ref,
                bout_ref, out_ref, s_acc):
    i = pl.program_id(0)
    part = part_ref[...]
    acc = jnp.concatenate([part[0, :, :HALF], part[1, :, :HALF]], axis=1)
    yb = y_ref[...]
    y = jnp.concatenate([yb[0, :, :HALF], yb[1, :, :HALF]], axis=1)
    deg = deg_ref[...]
    degsum = deg[0, 0] + deg[1, 0]
    dinv = lax.rsqrt(degsum + 1.0)
    h = (acc + y) * dinv + b1_ref[...]
    h = jnp.maximum(h, 0.0)
    rows = lax.broadcasted_iota(jnp.int32, (BLK, 1), 0) + i * BLK
    mask = rows < N
    h = jnp.where(mask, h, 0.0)
    b = batch_ref[...][0]
    oh = jnp.where(b == lax.broadcasted_iota(jnp.int32, (BLK, G), 1),
                   1.0, 0.0)
    hext = jnp.concatenate([h, mask.astype(jnp.float32)], axis=1)
    partial = lax.dot_general(oh, hext, (((0,), (0,)), ((), ())),
                              preferred_element_type=jnp.float32,
                              precision=lax.Precision.HIGHEST)
    prev = jnp.where(i == 0, jnp.zeros_like(partial), s_acc[...])
    tot = prev + partial
    s_acc[...] = tot

    @pl.when(i == NB - 1)
    def _():
        cnt = tot[:, FEAT:FEAT + 1]
        pooled = tot[:, :FEAT] / jnp.maximum(cnt, 1.0)
        logits = jnp.dot(pooled, wout_ref[...],
                         preferred_element_type=jnp.float32,
                         precision=lax.Precision.HIGHEST) + bout_ref[...]
        m = jnp.max(logits, axis=1, keepdims=True)
        ex = jnp.exp(logits - m)
        out_ref[...] = ex / jnp.sum(ex, axis=1, keepdims=True)


def _final_call(part, y2, deg4, batch3, b1, wout, bout):
    return pl.pallas_call(
        _final_body,
        grid=(NB,),
        in_specs=[
            pl.BlockSpec((2, BLK, HALFW), lambda i: (0, i, 0)),
            pl.BlockSpec((2, BLK, HALFW), lambda i: (0, i, 0)),
            pl.BlockSpec((2, 1, BLK, 1), lambda i: (0, i, 0, 0)),
            pl.BlockSpec((1, BLK, 1), lambda i: (i, 0, 0)),
            pl.BlockSpec((1, FEAT), lambda i: (0, 0)),
            pl.BlockSpec((FEAT, OUT), lambda i: (0, 0)),
            pl.BlockSpec((1, OUT), lambda i: (0, 0)),
        ],
        out_specs=pl.BlockSpec((G, OUT), lambda i: (0, 0)),
        out_shape=jax.ShapeDtypeStruct((G, OUT), jnp.float32),
        scratch_shapes=[pltpu.VMEM((G, FEAT + 1), jnp.float32)],
    )(part, y2, deg4, batch3, b1, wout, bout)


def kernel(prot_x, prot_edge_index, prot_dist, prot_batch, W1, b1, W_out,
           b_out):
    del prot_dist
    src2 = prot_edge_index[0].reshape(NCH, CHUNK)
    dst2 = prot_edge_index[1].reshape(NCH, CHUNK)
    zeros1 = jnp.zeros((N_PAD, 1), jnp.float32)
    ones1 = jnp.ones((CHUNK, 1), jnp.float32)
    zeros16 = jnp.zeros((N_PAD, HALFW), jnp.float32)

    deg4 = _deg_call(dst2, zeros1, ones1)
    y2 = _y_call(prot_x, W1, deg4)
    part = _scat_call(src2, dst2, y2, zeros16)

    batch3 = jnp.concatenate(
        [prot_batch, jnp.full((N_PAD - N,), G, jnp.int32)]).reshape(NB, BLK, 1)
    return _final_call(part, y2, deg4, batch3, b1.reshape(1, FEAT), W_out,
                       b_out.reshape(1, OUT))


# speedup vs baseline: 70.2932x; 1.0263x over previous
import jax
import jax.numpy as jnp
from jax import lax
from jax.experimental import pallas as pl
from jax.experimental.pallas import tpu as pltpu
from jax.experimental.pallas import tpu_sc as plsc

N = 100000
E = 3200000
FEAT = 20
HALF = FEAT // 2
HALFW = 16
G = 64
OUT = 5

BLK = 6272
NB = 16
N_PAD = NB * BLK
CHUNK = 128
K_SUB = 8
NCH = E // CHUNK
NSUP = NCH // K_SUB
ROWS_PER_TILE = N_PAD // 16

_SC_PARAMS = pltpu.CompilerParams(use_tc_tiling_on_sc=False)


def _mesh():
    return plsc.VectorSubcoreMesh(core_axis_name="c", subcore_axis_name="s")


def _half_range(t):
    q, r = NSUP // 16, NSUP % 16
    base = t * q + jnp.minimum(t, r)
    n = q + jnp.where(t < r, 1, 0)
    return base, n


def _split_range(c, t):
    q, r = NSUP // 32, NSUP % 32
    w = c * 16 + t
    base = w * q + jnp.minimum(w, r)
    n = q + jnp.where(w < r, 1, 0)
    return base, n


def _deg_body(dst2_ref, zeros1_ref, out_ref, deg_sp, idx_v, ones_v):
    c = lax.axis_index("c")
    t = lax.axis_index("s")
    r0 = t * ROWS_PER_TILE
    for k in range(CHUNK // 16):
        ones_v[pl.ds(k * 16, 16)] = jnp.ones((16,), jnp.float32)
    pltpu.sync_copy(zeros1_ref.at[pl.ds(r0, ROWS_PER_TILE)],
                    deg_sp.at[pl.ds(r0, ROWS_PER_TILE)])
    plsc.subcore_barrier()
    base, n = _split_range(c, t)

    def sbody(s, carry):
        b8 = (base + s) * K_SUB
        pltpu.sync_copy(dst2_ref.at[pl.ds(b8, K_SUB)], idx_v)
        for j in range(K_SUB):
            pltpu.sync_copy(ones_v, deg_sp.at[idx_v.at[j]], add=True)
        return carry

    lax.fori_loop(0, n, sbody, 0)
    plsc.subcore_barrier()
    pltpu.sync_copy(deg_sp.at[pl.ds(r0, ROWS_PER_TILE)],
                    out_ref.at[c, pl.ds(r0, ROWS_PER_TILE)])


def _deg_call(dst2, zeros1):
    return pl.kernel(
        _deg_body,
        out_type=jax.ShapeDtypeStruct((2, N_PAD), jnp.float32),
        mesh=_mesh(),
        compiler_params=_SC_PARAMS,
        scratch_types=[
            pltpu.VMEM_SHARED((N_PAD,), jnp.float32),
            pltpu.VMEM((K_SUB, CHUNK), jnp.int32),
            pltpu.VMEM((CHUNK,), jnp.float32),
        ],
    )(dst2, zeros1)


K_E = 4
NSUP_E = NCH // K_E
NBUF = 3


def _scat_body(src2_ref, dst2_ref, y2_ref, zeros16_ref, part_ref,
               acc_sp, sidx_v, didx_v, rows_v,
               sg0, sg1, sg2, ss0, ss1, ss2, si0, si1, si2, sd):
    c = lax.axis_index("c")
    t = lax.axis_index("s")
    r0 = t * ROWS_PER_TILE
    pltpu.sync_copy(zeros16_ref.at[pl.ds(r0, ROWS_PER_TILE)],
                    acc_sp.at[pl.ds(r0, ROWS_PER_TILE)])
    plsc.subcore_barrier()
    q, r = NSUP_E // 16, NSUP_E % 16
    base = t * q + jnp.minimum(t, r)
    n = q + jnp.where(t < r, 1, 0)
    sg = (sg0, sg1, sg2)
    ss = (ss0, ss1, ss2)
    si = (si0, si1, si2)

    def fire_sidx(su, b):
        pltpu.async_copy(src2_ref.at[pl.ds(su * K_E, K_E)], sidx_v.at[b],
                         si[b])

    def fire_didx(su, b):
        pltpu.async_copy(dst2_ref.at[pl.ds(su * K_E, K_E)], didx_v.at[b], sd)

    def fire_gathers(su, b):
        pltpu.make_async_copy(src2_ref.at[pl.ds(0, K_E)], sidx_v.at[b],
                              si[b]).wait()
        for j in range(K_E):
            pltpu.async_copy(y2_ref.at[c].at[sidx_v.at[b].at[j]],
                             rows_v.at[b, j], sg[b])

    def wait_gathers(b):
        for j in range(K_E):
            pltpu.make_async_copy(zeros16_ref.at[pl.ds(0, CHUNK)],
                                  rows_v.at[b, j], sg[b]).wait()

    def wait_didx(b):
        pltpu.make_async_copy(src2_ref.at[pl.ds(0, K_E)], didx_v.at[b],
                              sd).wait()

    def fire_scatter(b):
        for j in range(K_E):
            pltpu.async_copy(rows_v.at[b, j], acc_sp.at[didx_v.at[b].at[j]],
                             ss[b], add=True)

    def drain_scatter(b):
        for j in range(K_E):
            pltpu.make_async_copy(zeros16_ref.at[pl.ds(0, CHUNK)],
                                  rows_v.at[b, j], ss[b]).wait()

    fire_sidx(base, 0)
    fire_sidx(base + 1, 1)
    fire_sidx(base + 2, 2)
    fire_gathers(base, 0)
    fire_gathers(base + 1, 1)

    def sbody(s, carry):
        for P in range(NBUF):
            @pl.when(lax.rem(s, NBUF) == P)
            def _(P=P):
                @pl.when(s >= 1)
                def _():
                    drain_scatter((P + 2) % NBUF)

                fire_didx(base + s, P)

                @pl.when(s + 2 <= n - 1)
                def _():
                    fire_gathers(base + s + 2, (P + 2) % NBUF)

                wait_gathers(P)
                wait_didx(P)
                fire_scatter(P)

                @pl.when(s + 3 <= n - 1)
                def _():
                    fire_sidx(base + s + 3, P)
        return carry

    lax.fori_loop(0, n, sbody, 0)
    for P in range(NBUF):
        @pl.when(lax.rem(n - 1, NBUF) == P)
        def _(P=P):
            drain_scatter(P)
    plsc.subcore_barrier()
    pltpu.sync_copy(acc_sp.at[pl.ds(r0, ROWS_PER_TILE)],
                    part_ref.at[c, pl.ds(r0, ROWS_PER_TILE)])


def _scat_call(src2, dst2, y2, zeros16):
    return pl.kernel(
        _scat_body,
        out_type=jax.ShapeDtypeStruct((2, N_PAD, HALFW), jnp.float32),
        mesh=_mesh(),
        compiler_params=_SC_PARAMS,
        scratch_types=[
            pltpu.VMEM_SHARED((N_PAD, HALFW), jnp.float32),
            pltpu.VMEM((NBUF, K_E, CHUNK), jnp.int32),
            pltpu.VMEM((NBUF, K_E, CHUNK), jnp.int32),
            pltpu.VMEM((NBUF, K_E, CHUNK, HALFW), jnp.float32),
        ] + [pltpu.SemaphoreType.DMA] * 10,
    )(src2, dst2, y2, zeros16)


def _y_body(x_ref, w1_ref, deg_ref, y_ref):
    xw = jnp.dot(x_ref[...], w1_ref[...], preferred_element_type=jnp.float32,
                 precision=lax.Precision.HIGHEST)
    deg = deg_ref[...]
    degsum = deg[0, 0] + deg[1, 0]
    dinv = lax.rsqrt(degsum + 1.0)
    y = xw * dinv
    zpad = jnp.zeros((BLK, HALFW - HALF), jnp.float32)
    y_ref[0] = jnp.concatenate([y[:, :HALF], zpad], axis=1)
    y_ref[1] = jnp.concatenate([y[:, HALF:], zpad], axis=1)


def _y_call(x, w1, deg4):
    return pl.pallas_call(
        _y_body,
        grid=(NB,),
        in_specs=[
            pl.BlockSpec((BLK, FEAT), lambda i: (i, 0)),
            pl.BlockSpec((FEAT, FEAT), lambda i: (0, 0)),
            pl.BlockSpec((2, 1, BLK, 1), lambda i: (0, i, 0, 0)),
        ],
        out_specs=pl.BlockSpec((2, BLK, HALFW), lambda i: (0, i, 0)),
        out_shape=jax.ShapeDtypeStruct((2, N_PAD, HALFW), jnp.float32),
    )(x, w1, deg4)


def _final_body(part_ref, y_ref, deg_ref, batch_ref, b1_ref, wout_ref,
                bout_ref, out_ref, s_acc):
    i = pl.program_id(0)
    part = part_ref[...]
    acc = jnp.concatenate([part[0, :, :HALF], part[1, :, :HALF]], axis=1)
    yb = y_ref[...]
    y = jnp.concatenate([yb[0, :, :HALF], yb[1, :, :HALF]], axis=1)
    deg = deg_ref[...]
    degsum = deg[0, 0] + deg[1, 0]
    dinv = lax.rsqrt(degsum + 1.0)
    h = (acc + y) * dinv + b1_ref[...]
    h = jnp.maximum(h, 0.0)
    rows = lax.broadcasted_iota(jnp.int32, (BLK, 1), 0) + i * BLK
    mask = rows < N
    h = jnp.where(mask, h, 0.0)
    b = batch_ref[...][0]
    oh = jnp.where(b == lax.broadcasted_iota(jnp.int32, (BLK, G), 1),
                   1.0, 0.0)
    hext = jnp.concatenate([h, mask.astype(jnp.float32)], axis=1)
    partial = lax.dot_general(oh, hext, (((0,), (0,)), ((), ())),
                              preferred_element_type=jnp.float32,
                              precision=lax.Precision.HIGHEST)
    prev = jnp.where(i == 0, jnp.zeros_like(partial), s_acc[...])
    tot = prev + partial
    s_acc[...] = tot

    @pl.when(i == NB - 1)
    def _():
        cnt = tot[:, FEAT:FEAT + 1]
        pooled = tot[:, :FEAT] / jnp.maximum(cnt, 1.0)
        logits = jnp.dot(pooled, wout_ref[...],
                         preferred_element_type=jnp.float32,
                         precision=lax.Precision.HIGHEST) + bout_ref[...]
        m = jnp.max(logits, axis=1, keepdims=True)
        ex = jnp.exp(logits - m)
        out_ref[...] = ex / jnp.sum(ex, axis=1, keepdims=True)


def _final_call(part, y2, deg4, batch3, b1, wout, bout):
    return pl.pallas_call(
        _final_body,
        grid=(NB,),
        in_specs=[
            pl.BlockSpec((2, BLK, HALFW), lambda i: (0, i, 0)),
            pl.BlockSpec((2, BLK, HALFW), lambda i: (0, i, 0)),
            pl.BlockSpec((2, 1, BLK, 1), lambda i: (0, i, 0, 0)),
            pl.BlockSpec((1, BLK, 1), lambda i: (i, 0, 0)),
            pl.BlockSpec((1, FEAT), lambda i: (0, 0)),
            pl.BlockSpec((FEAT, OUT), lambda i: (0, 0)),
            pl.BlockSpec((1, OUT), lambda i: (0, 0)),
        ],
        out_specs=pl.BlockSpec((G, OUT), lambda i: (0, 0)),
        out_shape=jax.ShapeDtypeStruct((G, OUT), jnp.float32),
        scratch_shapes=[pltpu.VMEM((G, FEAT + 1), jnp.float32)],
    )(part, y2, deg4, batch3, b1, wout, bout)


def kernel(prot_x, prot_edge_index, prot_dist, prot_batch, W1, b1, W_out,
           b_out):
    del prot_dist
    src2 = prot_edge_index[0].reshape(NCH, CHUNK)
    dst2 = prot_edge_index[1].reshape(NCH, CHUNK)
    zeros1 = jnp.zeros((N_PAD,), jnp.float32)
    zeros16 = jnp.zeros((N_PAD, HALFW), jnp.float32)

    degp = _deg_call(dst2, zeros1)
    deg4 = degp.reshape(2, NB, BLK, 1)
    y2 = _y_call(prot_x, W1, deg4)
    part = _scat_call(src2, dst2, y2, zeros16)

    batch3 = jnp.concatenate(
        [prot_batch, jnp.full((N_PAD - N,), G, jnp.int32)]).reshape(NB, BLK, 1)
    return _final_call(part, y2, deg4, batch3, b1.reshape(1, FEAT), W_out,
                       b_out.reshape(1, OUT))


# speedup vs baseline: 75.5446x; 1.0747x over previous
import jax
import jax.numpy as jnp
from jax import lax
from jax.experimental import pallas as pl
from jax.experimental.pallas import tpu as pltpu
from jax.experimental.pallas import tpu_sc as plsc

N = 100000
E = 3200000
FEAT = 20
HALF = FEAT // 2
HALFW = 16
G = 64
OUT = 5

BLK = 6272
NB = 16
N_PAD = NB * BLK
CHUNK = 128
K_SUB = 8
NCH = E // CHUNK
NSUP = NCH // K_SUB
ROWS_PER_TILE = N_PAD // 16

_SC_PARAMS = pltpu.CompilerParams(use_tc_tiling_on_sc=False)


def _mesh():
    return plsc.VectorSubcoreMesh(core_axis_name="c", subcore_axis_name="s")


def _half_range(t):
    q, r = NSUP // 16, NSUP % 16
    base = t * q + jnp.minimum(t, r)
    n = q + jnp.where(t < r, 1, 0)
    return base, n


def _split_range(c, t):
    q, r = NSUP // 32, NSUP % 32
    w = c * 16 + t
    base = w * q + jnp.minimum(w, r)
    n = q + jnp.where(w < r, 1, 0)
    return base, n


def _deg_body(dst2_ref, zeros1_ref, out_ref, deg_sp, idx_v, ones_v,
              si0, si1, ssc):
    c = lax.axis_index("c")
    t = lax.axis_index("s")
    r0 = t * ROWS_PER_TILE
    for k in range(CHUNK // 16):
        ones_v[pl.ds(k * 16, 16)] = jnp.ones((16,), jnp.float32)
    pltpu.sync_copy(zeros1_ref.at[pl.ds(r0, ROWS_PER_TILE)],
                    deg_sp.at[pl.ds(r0, ROWS_PER_TILE)])
    plsc.subcore_barrier()
    base, n = _split_range(c, t)
    si = (si0, si1)

    def fire_idx(su, b):
        pltpu.async_copy(dst2_ref.at[pl.ds(su * K_SUB, K_SUB)], idx_v.at[b],
                         si[b])

    def wait_idx(b):
        pltpu.make_async_copy(dst2_ref.at[pl.ds(0, K_SUB)], idx_v.at[b],
                              si[b]).wait()

    def drain_adds():
        for j in range(K_SUB):
            pltpu.make_async_copy(dst2_ref.at[pl.ds(0, 1)], ones_v,
                                  ssc).wait()

    fire_idx(base, 0)

    def sbody(s, carry):
        for P in range(2):
            @pl.when(lax.rem(s, 2) == P)
            def _(P=P):
                wait_idx(P)
                for j in range(K_SUB):
                    pltpu.async_copy(ones_v, deg_sp.at[idx_v.at[P].at[j]],
                                     ssc, add=True)

                @pl.when(s >= 1)
                def _():
                    drain_adds()

                @pl.when(s + 1 <= n - 1)
                def _():
                    fire_idx(base + s + 1, 1 - P)
        return carry

    lax.fori_loop(0, n, sbody, 0)
    drain_adds()
    plsc.subcore_barrier()
    pltpu.sync_copy(deg_sp.at[pl.ds(r0, ROWS_PER_TILE)],
                    out_ref.at[c, pl.ds(r0, ROWS_PER_TILE)])


def _deg_call(dst2, zeros1):
    return pl.kernel(
        _deg_body,
        out_type=jax.ShapeDtypeStruct((2, N_PAD), jnp.float32),
        mesh=_mesh(),
        compiler_params=_SC_PARAMS,
        scratch_types=[
            pltpu.VMEM_SHARED((N_PAD,), jnp.float32),
            pltpu.VMEM((2, K_SUB, CHUNK), jnp.int32),
            pltpu.VMEM((CHUNK,), jnp.float32),
            pltpu.SemaphoreType.DMA,
            pltpu.SemaphoreType.DMA,
            pltpu.SemaphoreType.DMA,
        ],
    )(dst2, zeros1)


K_E = 4
NSUP_E = NCH // K_E
NBUF = 3


def _scat_body(src2_ref, dst2_ref, y2_ref, zeros16_ref, part_ref,
               acc_sp, sidx_v, didx_v, rows_v,
               sg0, sg1, sg2, ss0, ss1, ss2, si0, si1, si2, sd):
    c = lax.axis_index("c")
    t = lax.axis_index("s")
    r0 = t * ROWS_PER_TILE
    pltpu.sync_copy(zeros16_ref.at[pl.ds(r0, ROWS_PER_TILE)],
                    acc_sp.at[pl.ds(r0, ROWS_PER_TILE)])
    plsc.subcore_barrier()
    q, r = NSUP_E // 16, NSUP_E % 16
    base = t * q + jnp.minimum(t, r)
    n = q + jnp.where(t < r, 1, 0)
    sg = (sg0, sg1, sg2)
    ss = (ss0, ss1, ss2)
    si = (si0, si1, si2)

    def fire_sidx(su, b):
        pltpu.async_copy(src2_ref.at[pl.ds(su * K_E, K_E)], sidx_v.at[b],
                         si[b])

    def fire_didx(su, b):
        pltpu.async_copy(dst2_ref.at[pl.ds(su * K_E, K_E)], didx_v.at[b], sd)

    def fire_gathers(su, b):
        pltpu.make_async_copy(src2_ref.at[pl.ds(0, K_E)], sidx_v.at[b],
                              si[b]).wait()
        for j in range(K_E):
            pltpu.async_copy(y2_ref.at[c].at[sidx_v.at[b].at[j]],
                             rows_v.at[b, j], sg[b])

    def wait_gathers(b):
        for j in range(K_E):
            pltpu.make_async_copy(zeros16_ref.at[pl.ds(0, CHUNK)],
                                  rows_v.at[b, j], sg[b]).wait()

    def wait_didx(b):
        pltpu.make_async_copy(src2_ref.at[pl.ds(0, K_E)], didx_v.at[b],
                              sd).wait()

    def fire_scatter(b):
        for j in range(K_E):
            pltpu.async_copy(rows_v.at[b, j], acc_sp.at[didx_v.at[b].at[j]],
                             ss[b], add=True)

    def drain_scatter(b):
        for j in range(K_E):
            pltpu.make_async_copy(zeros16_ref.at[pl.ds(0, CHUNK)],
                                  rows_v.at[b, j], ss[b]).wait()

    fire_sidx(base, 0)
    fire_sidx(base + 1, 1)
    fire_sidx(base + 2, 2)
    fire_gathers(base, 0)
    fire_gathers(base + 1, 1)

    def sbody(s, carry):
        for P in range(NBUF):
            @pl.when(lax.rem(s, NBUF) == P)
            def _(P=P):
                @pl.when(s >= 1)
                def _():
                    drain_scatter((P + 2) % NBUF)

                fire_didx(base + s, P)

                @pl.when(s + 2 <= n - 1)
                def _():
                    fire_gathers(base + s + 2, (P + 2) % NBUF)

                wait_gathers(P)
                wait_didx(P)
                fire_scatter(P)

                @pl.when(s + 3 <= n - 1)
                def _():
                    fire_sidx(base + s + 3, P)
        return carry

    lax.fori_loop(0, n, sbody, 0)
    for P in range(NBUF):
        @pl.when(lax.rem(n - 1, NBUF) == P)
        def _(P=P):
            drain_scatter(P)
    plsc.subcore_barrier()
    pltpu.sync_copy(acc_sp.at[pl.ds(r0, ROWS_PER_TILE)],
                    part_ref.at[c, pl.ds(r0, ROWS_PER_TILE)])


def _scat_call(src2, dst2, y2, zeros16):
    return pl.kernel(
        _scat_body,
        out_type=jax.ShapeDtypeStruct((2, N_PAD, HALFW), jnp.float32),
        mesh=_mesh(),
        compiler_params=_SC_PARAMS,
        scratch_types=[
            pltpu.VMEM_SHARED((N_PAD, HALFW), jnp.float32),
            pltpu.VMEM((NBUF, K_E, CHUNK), jnp.int32),
            pltpu.VMEM((NBUF, K_E, CHUNK), jnp.int32),
            pltpu.VMEM((NBUF, K_E, CHUNK, HALFW), jnp.float32),
        ] + [pltpu.SemaphoreType.DMA] * 10,
    )(src2, dst2, y2, zeros16)


def _y_body(x_ref, w1_ref, deg_ref, y_ref):
    xw = jnp.dot(x_ref[...], w1_ref[...], preferred_element_type=jnp.float32,
                 precision=lax.Precision.HIGHEST)
    deg = deg_ref[...]
    degsum = deg[0, 0] + deg[1, 0]
    dinv = lax.rsqrt(degsum + 1.0)
    y = xw * dinv
    zpad = jnp.zeros((BLK, HALFW - HALF), jnp.float32)
    y_ref[0] = jnp.concatenate([y[:, :HALF], zpad], axis=1)
    y_ref[1] = jnp.concatenate([y[:, HALF:], zpad], axis=1)


def _y_call(x, w1, deg4):
    return pl.pallas_call(
        _y_body,
        grid=(NB,),
        in_specs=[
            pl.BlockSpec((BLK, FEAT), lambda i: (i, 0)),
            pl.BlockSpec((FEAT, FEAT), lambda i: (0, 0)),
            pl.BlockSpec((2, 1, BLK, 1), lambda i: (0, i, 0, 0)),
        ],
        out_specs=pl.BlockSpec((2, BLK, HALFW), lambda i: (0, i, 0)),
        out_shape=jax.ShapeDtypeStruct((2, N_PAD, HALFW), jnp.float32),
    )(x, w1, deg4)


def _final_body(part_ref, y_ref, deg_ref, batch_ref, b1_ref, wout_ref,
                bout_ref, out_ref, s_acc):
    i = pl.program_id(0)
    part = part_ref[...]
    acc = jnp.concatenate([part[0, :, :HALF], part[1, :, :HALF]], axis=1)
    yb = y_ref[...]
    y = jnp.concatenate([yb[0, :, :HALF], yb[1, :, :HALF]], axis=1)
    deg = deg_ref[...]
    degsum = deg[0, 0] + deg[1, 0]
    dinv = lax.rsqrt(degsum + 1.0)
    h = (acc + y) * dinv + b1_ref[...]
    h = jnp.maximum(h, 0.0)
    rows = lax.broadcasted_iota(jnp.int32, (BLK, 1), 0) + i * BLK
    mask = rows < N
    h = jnp.where(mask, h, 0.0)
    b = batch_ref[...][0]
    oh = jnp.where(b == lax.broadcasted_iota(jnp.int32, (BLK, G), 1),
                   1.0, 0.0)
    hext = jnp.concatenate([h, mask.astype(jnp.float32)], axis=1)
    partial = lax.dot_general(oh, hext, (((0,), (0,)), ((), ())),
                              preferred_element_type=jnp.float32,
                              precision=lax.Precision.HIGHEST)
    prev = jnp.where(i == 0, jnp.zeros_like(partial), s_acc[...])
    tot = prev + partial
    s_acc[...] = tot

    @pl.when(i == NB - 1)
    def _():
        cnt = tot[:, FEAT:FEAT + 1]
        pooled = tot[:, :FEAT] / jnp.maximum(cnt, 1.0)
        logits = jnp.dot(pooled, wout_ref[...],
                         preferred_element_type=jnp.float32,
                         precision=lax.Precision.HIGHEST) + bout_ref[...]
        m = jnp.max(logits, axis=1, keepdims=True)
        ex = jnp.exp(logits - m)
        out_ref[...] = ex / jnp.sum(ex, axis=1, keepdims=True)


def _final_call(part, y2, deg4, batch3, b1, wout, bout):
    return pl.pallas_call(
        _final_body,
        grid=(NB,),
        in_specs=[
            pl.BlockSpec((2, BLK, HALFW), lambda i: (0, i, 0)),
            pl.BlockSpec((2, BLK, HALFW), lambda i: (0, i, 0)),
            pl.BlockSpec((2, 1, BLK, 1), lambda i: (0, i, 0, 0)),
            pl.BlockSpec((1, BLK, 1), lambda i: (i, 0, 0)),
            pl.BlockSpec((1, FEAT), lambda i: (0, 0)),
            pl.BlockSpec((FEAT, OUT), lambda i: (0, 0)),
            pl.BlockSpec((1, OUT), lambda i: (0, 0)),
        ],
        out_specs=pl.BlockSpec((G, OUT), lambda i: (0, 0)),
        out_shape=jax.ShapeDtypeStruct((G, OUT), jnp.float32),
        scratch_shapes=[pltpu.VMEM((G, FEAT + 1), jnp.float32)],
    )(part, y2, deg4, batch3, b1, wout, bout)


def kernel(prot_x, prot_edge_index, prot_dist, prot_batch, W1, b1, W_out,
           b_out):
    del prot_dist
    src2 = prot_edge_index[0].reshape(NCH, CHUNK)
    dst2 = prot_edge_index[1].reshape(NCH, CHUNK)
    zeros1 = jnp.zeros((N_PAD,), jnp.float32)
    zeros16 = jnp.zeros((N_PAD, HALFW), jnp.float32)

    degp = _deg_call(dst2, zeros1)
    deg4 = degp.reshape(2, NB, BLK, 1)
    y2 = _y_call(prot_x, W1, deg4)
    part = _scat_call(src2, dst2, y2, zeros16)

    batch3 = jnp.concatenate(
        [prot_batch, jnp.full((N_PAD - N,), G, jnp.int32)]).reshape(NB, BLK, 1)
    return _final_call(part, y2, deg4, batch3, b1.reshape(1, FEAT), W_out,
                       b_out.reshape(1, OUT))
